# baseline (speedup 1.0000x reference)
import functools
import math

import jax
import jax.numpy as jnp
from jax import lax
from jax.experimental import pallas as pl
from jax.experimental.pallas import tpu as pltpu
from jax.experimental.pallas import tpu_sc as plsc

D_MODEL = 64
LANES = 16
NUM_WORKERS = 32
BLK = 128


def _emb_body(n_planes, scale,
              xt_hbm, tp_hbm, out_hbm,
              idx_v, idx2_v, h64_v, rows_v, outbuf_v, gsem):
    wid = lax.axis_index("s") * 2 + lax.axis_index("c")
    b0_base = wid * BLK

    def plane_body(b1, carry):
        pltpu.sync_copy(xt_hbm.at[b1, pl.ds(b0_base, BLK)], idx_v)
        for k in range(BLK // LANES):
            sl = pl.ds(k * LANES, LANES)
            t = idx_v[sl]
            idx2_v[sl] = lax.shift_right_logical(t, 1)
            h64_v[sl] = lax.mul(lax.bitwise_and(t, 1), D_MODEL)
        pltpu.async_copy(tp_hbm.at[idx2_v], rows_v, gsem).wait()

        def group_body(g, c2):
            base16 = g * LANES
            sl = pl.ds(base16, LANES)
            hv = h64_v[sl]
            tok = lax.iota(jnp.int32, LANES) + base16
            for d in range(D_MODEL):
                v = plsc.load_gather(rows_v, [tok, hv + d])
                outbuf_v[d, sl] = v * scale
            return c2

        lax.fori_loop(0, BLK // LANES, group_body, 0)
        pltpu.sync_copy(outbuf_v, out_hbm.at[b1, :, pl.ds(b0_base, BLK)])
        return carry

    lax.fori_loop(0, n_planes, plane_body, 0)


def kernel(x, table):
    n_b0, n_b1 = x.shape
    vocab = table.shape[0]
    scale = math.sqrt(D_MODEL)

    xt = x.astype(jnp.int32).T
    tp = table.reshape(vocab // 2, 2 * D_MODEL)

    mesh = plsc.VectorSubcoreMesh(core_axis_name="c", subcore_axis_name="s")

    emb = pl.kernel(
        functools.partial(_emb_body, n_b1, scale),
        out_type=jax.ShapeDtypeStruct((n_b1, D_MODEL, n_b0), jnp.float32),
        mesh=mesh,
        scratch_types=[
            pltpu.VMEM((BLK,), jnp.int32),
            pltpu.VMEM((BLK,), jnp.int32),
            pltpu.VMEM((BLK,), jnp.int32),
            pltpu.VMEM((BLK, 2 * D_MODEL), jnp.float32),
            pltpu.VMEM((D_MODEL, BLK), jnp.float32),
            pltpu.SemaphoreType.DMA,
        ],
        compiler_params=pltpu.CompilerParams(needs_layout_passes=False),
    )
    out_t = emb(xt, tp)
    return out_t.transpose(2, 0, 1)


# speedup vs baseline: 1.7114x; 1.7114x over previous
import functools
import math

import jax
import jax.numpy as jnp
from jax import lax
from jax.experimental import pallas as pl
from jax.experimental.pallas import tpu as pltpu
from jax.experimental.pallas import tpu_sc as plsc

D_MODEL = 64
LANES = 16
NUM_WORKERS = 32
BLK = 128
N_GRP = BLK // LANES


def _emb_body(n_planes, scale,
              xt_hbm, tp_hbm, out_hbm,
              ix_a, ix_b, id2_a, id2_b, h_a, h_b,
              rows_a, rows_b, o_a, o_b, sink,
              gsem, osem_a, osem_b):
    wid = lax.axis_index("s") * 2 + lax.axis_index("c")
    b0_base = wid * BLK
    n_pairs = n_planes // 2

    def idx_fetch(ix, b1):
        pltpu.sync_copy(xt_hbm.at[b1, pl.ds(b0_base, BLK)], ix)

    def prep(ix, id2, h):
        for k in range(N_GRP):
            sl = pl.ds(k * LANES, LANES)
            t = ix[sl]
            id2[sl] = lax.shift_right_logical(t, 1)
            h[sl] = lax.mul(lax.bitwise_and(t, 1), D_MODEL)

    def gather_start(id2, rows):
        pltpu.make_async_copy(tp_hbm.at[id2], rows, gsem).start()

    def gather_wait(id2, rows):
        pltpu.make_async_copy(tp_hbm.at[id2], rows, gsem).wait()

    def transpose(rows, h, o):
        lane = lax.iota(jnp.int32, LANES)

        def group_body(g, carry):
            hv = h[pl.ds(g * LANES, LANES)]
            tok = lane + g * LANES
            sl = pl.ds(lax.mul(g, LANES), LANES)

            def d_body(d, col):
                v = plsc.load_gather(rows, [tok, col])
                o[d, sl] = v * scale
                return col + 1

            fcol = plsc.parallel_loop(
                0, D_MODEL, unroll=8, carry=hv)(d_body)
            sink[sl] = fcol
            return carry

        lax.fori_loop(0, N_GRP, group_body, 0)

    def out_start(o, b1, osem):
        pltpu.make_async_copy(
            o, out_hbm.at[b1, :, pl.ds(b0_base, BLK)], osem).start()

    def out_wait(o, b1, osem):
        pltpu.make_async_copy(
            o, out_hbm.at[b1, :, pl.ds(b0_base, BLK)], osem).wait()

    idx_fetch(ix_a, 0)
    prep(ix_a, id2_a, h_a)
    gather_start(id2_a, rows_a)

    def pair_body(p, carry):
        a = 2 * p
        b = a + 1
        idx_fetch(ix_b, b)
        prep(ix_b, id2_b, h_b)
        gather_wait(id2_a, rows_a)
        gather_start(id2_b, rows_b)

        @pl.when(p > 0)
        def _():
            out_wait(o_a, a - 2, osem_a)

        transpose(rows_a, h_a, o_a)
        out_start(o_a, a, osem_a)

        @pl.when(p + 1 < n_pairs)
        def _():
            idx_fetch(ix_a, a + 2)
            prep(ix_a, id2_a, h_a)

        gather_wait(id2_b, rows_b)

        @pl.when(p + 1 < n_pairs)
        def _():
            gather_start(id2_a, rows_a)

        @pl.when(p > 0)
        def _():
            out_wait(o_b, b - 2, osem_b)

        transpose(rows_b, h_b, o_b)
        out_start(o_b, b, osem_b)
        return carry

    lax.fori_loop(0, n_pairs, pair_body, 0)
    out_wait(o_a, n_planes - 2, osem_a)
    out_wait(o_b, n_planes - 1, osem_b)


def kernel(x, table):
    n_b0, n_b1 = x.shape
    vocab = table.shape[0]
    scale = math.sqrt(D_MODEL)

    xt = x.astype(jnp.int32).T
    tp = table.reshape(vocab // 2, 2 * D_MODEL)

    mesh = plsc.VectorSubcoreMesh(core_axis_name="c", subcore_axis_name="s")

    emb = pl.kernel(
        functools.partial(_emb_body, n_b1, scale),
        out_type=jax.ShapeDtypeStruct((n_b1, D_MODEL, n_b0), jnp.float32),
        mesh=mesh,
        scratch_types=[
            pltpu.VMEM((BLK,), jnp.int32),
            pltpu.VMEM((BLK,), jnp.int32),
            pltpu.VMEM((BLK,), jnp.int32),
            pltpu.VMEM((BLK,), jnp.int32),
            pltpu.VMEM((BLK,), jnp.int32),
            pltpu.VMEM((BLK,), jnp.int32),
            pltpu.VMEM((BLK, 2 * D_MODEL), jnp.float32),
            pltpu.VMEM((BLK, 2 * D_MODEL), jnp.float32),
            pltpu.VMEM((D_MODEL, BLK), jnp.float32),
            pltpu.VMEM((D_MODEL, BLK), jnp.float32),
            pltpu.VMEM((BLK,), jnp.int32),
            pltpu.SemaphoreType.DMA,
            pltpu.SemaphoreType.DMA,
            pltpu.SemaphoreType.DMA,
        ],
        compiler_params=pltpu.CompilerParams(needs_layout_passes=False),
    )
    out_t = emb(xt, tp)
    return out_t.transpose(2, 0, 1)


# speedup vs baseline: 1.8404x; 1.0754x over previous
import functools
import math

import jax
import jax.numpy as jnp
from jax import lax
from jax.experimental import pallas as pl
from jax.experimental.pallas import tpu as pltpu
from jax.experimental.pallas import tpu_sc as plsc

D_MODEL = 64
LANES = 16
NUM_WORKERS = 32
BLK = 128
N_GRP = BLK // LANES


def _emb_body(n_planes, scale,
              xt_hbm, tp_hbm, out_hbm,
              idq, hq, rows_a, rows_b, o_a, o_b, sink,
              gsem_a, gsem_b, osem_a, osem_b):
    wid = lax.axis_index("s") * 2 + lax.axis_index("c")
    b0_base = wid * BLK

    pltpu.sync_copy(xt_hbm.at[:, pl.ds(b0_base, BLK)], idq)

    def prep_body(p, carry):
        for k in range(N_GRP):
            sl = pl.ds(k * LANES, LANES)
            t = idq[p, sl]
            idq[p, sl] = lax.shift_right_logical(t, 1)
            hq[p, sl] = lax.mul(lax.bitwise_and(t, 1), D_MODEL)
        return carry

    lax.fori_loop(0, n_planes, prep_body, 0)

    def gather_start(b1, rows, gsem):
        pltpu.make_async_copy(tp_hbm.at[idq.at[b1]], rows, gsem).start()

    def gather_wait(b1, rows, gsem):
        pltpu.make_async_copy(tp_hbm.at[idq.at[b1]], rows, gsem).wait()

    def transpose(rows, b1, o):
        lane = lax.iota(jnp.int32, LANES)

        def group_body(g, carry):
            hv = hq[b1, pl.ds(g * LANES, LANES)]
            tok = lane + g * LANES
            sl = pl.ds(lax.mul(g, LANES), LANES)

            def d_body(d, col):
                v = plsc.load_gather(rows, [tok, col])
                o[d, sl] = v * scale
                return col + 1

            fcol = plsc.parallel_loop(
                0, D_MODEL, unroll=8, carry=hv)(d_body)
            sink[sl] = fcol
            return carry

        lax.fori_loop(0, N_GRP, group_body, 0)

    def out_start(o, b1, osem):
        pltpu.make_async_copy(
            o, out_hbm.at[b1, :, pl.ds(b0_base, BLK)], osem).start()

    def out_wait(o, b1, osem):
        pltpu.make_async_copy(
            o, out_hbm.at[b1, :, pl.ds(b0_base, BLK)], osem).wait()

    gather_start(0, rows_a, gsem_a)
    gather_start(1, rows_b, gsem_b)

    def pair_body(p, carry):
        a = 2 * p
        b = a + 1

        gather_wait(a, rows_a, gsem_a)

        @pl.when(p > 0)
        def _():
            out_wait(o_a, a - 2, osem_a)

        transpose(rows_a, a, o_a)
        out_start(o_a, a, osem_a)

        @pl.when(a + 2 < n_planes)
        def _():
            gather_start(a + 2, rows_a, gsem_a)

        gather_wait(b, rows_b, gsem_b)

        @pl.when(p > 0)
        def _():
            out_wait(o_b, b - 2, osem_b)

        transpose(rows_b, b, o_b)
        out_start(o_b, b, osem_b)

        @pl.when(b + 2 < n_planes)
        def _():
            gather_start(b + 2, rows_b, gsem_b)

        return carry

    lax.fori_loop(0, n_planes // 2, pair_body, 0)
    out_wait(o_a, n_planes - 2, osem_a)
    out_wait(o_b, n_planes - 1, osem_b)


def kernel(x, table):
    n_b0, n_b1 = x.shape
    vocab = table.shape[0]
    scale = math.sqrt(D_MODEL)

    xt = x.astype(jnp.int32).T
    tp = table.reshape(vocab // 2, 2 * D_MODEL)

    mesh = plsc.VectorSubcoreMesh(core_axis_name="c", subcore_axis_name="s")

    emb = pl.kernel(
        functools.partial(_emb_body, n_b1, scale),
        out_type=jax.ShapeDtypeStruct((n_b1, D_MODEL, n_b0), jnp.float32),
        mesh=mesh,
        scratch_types=[
            pltpu.VMEM((n_b1, BLK), jnp.int32),
            pltpu.VMEM((n_b1, BLK), jnp.int32),
            pltpu.VMEM((BLK, 2 * D_MODEL), jnp.float32),
            pltpu.VMEM((BLK, 2 * D_MODEL), jnp.float32),
            pltpu.VMEM((D_MODEL, BLK), jnp.float32),
            pltpu.VMEM((D_MODEL, BLK), jnp.float32),
            pltpu.VMEM((BLK,), jnp.int32),
            pltpu.SemaphoreType.DMA,
            pltpu.SemaphoreType.DMA,
            pltpu.SemaphoreType.DMA,
            pltpu.SemaphoreType.DMA,
        ],
        compiler_params=pltpu.CompilerParams(needs_layout_passes=False),
    )
    out_t = emb(xt, tp)
    return out_t.transpose(2, 0, 1)
